# baseline (speedup 1.0000x reference)
import jax
import jax.numpy as jnp
from jax import lax
from jax.experimental import pallas as pl
from jax.experimental.pallas import tpu as pltpu
from jax.experimental.pallas import tpu_sc as plsc

BATCH = 16384
IN_COLS = 100
OUT_COLS = 380
NUM_NONCAT = 60
_CAT_GROUPS = ((4, 60, 20, 60), (8, 80, 10, 140), (16, 90, 10, 220))

NUM_WORKERS = 32
ROWS_PER_WORKER = BATCH // NUM_WORKERS
CHUNK_ROWS = 64
CHUNKS = ROWS_PER_WORKER // CHUNK_ROWS
BLOCKS = CHUNK_ROWS // 16


def _block_body(in_v, out_v, chunk, blk):
    rows_l = blk * 16 + lax.iota(jnp.int32, 16)
    rows_g = chunk * CHUNK_ROWS + rows_l
    for c in range(NUM_NONCAT):
        col = jnp.full((16,), c, jnp.int32)
        v = plsc.load_gather(in_v, [rows_g, col])
        plsc.store_scatter(out_v, [rows_l, col], v)
    one = jnp.full((16,), 1.0, jnp.float32)
    zero = jnp.zeros((16,), jnp.float32)
    for card, src0, nparams, out0 in _CAT_GROUPS:
        for j in range(nparams):
            src = jnp.full((16,), src0 + j, jnp.int32)
            v = plsc.load_gather(in_v, [rows_g, src])
            for c in range(card):
                oh = jnp.where(v == float(c), one, zero)
                dst = jnp.full((16,), out0 + card * j + c, jnp.int32)
                plsc.store_scatter(out_v, [rows_l, dst], oh)


def _sc_kernel(x_hbm, out_hbm, in_v, out_v0, out_v1, sem0, sem1):
    wid = lax.axis_index("s") * 2 + lax.axis_index("c")
    row0 = wid * ROWS_PER_WORKER
    pltpu.sync_copy(x_hbm.at[pl.ds(row0, ROWS_PER_WORKER)], in_v)

    @pl.loop(0, CHUNKS, step=2)
    def _pair(c):
        for phase, (ob, sem) in enumerate(((out_v0, sem0), (out_v1, sem1))):
            chunk = c + phase

            @pl.when(chunk >= 2)
            def _wait_prev():
                pltpu.make_async_copy(
                    ob, out_hbm.at[pl.ds(row0, CHUNK_ROWS)], sem
                ).wait()

            @pl.loop(0, BLOCKS)
            def _blk(blk):
                _block_body(in_v, ob, chunk, blk)

            pltpu.async_copy(
                ob, out_hbm.at[pl.ds(row0 + chunk * CHUNK_ROWS, CHUNK_ROWS)], sem
            )

    pltpu.make_async_copy(out_v0, out_hbm.at[pl.ds(row0, CHUNK_ROWS)], sem0).wait()
    pltpu.make_async_copy(out_v1, out_hbm.at[pl.ds(row0, CHUNK_ROWS)], sem1).wait()


@jax.jit
def kernel(x):
    mesh = plsc.VectorSubcoreMesh(core_axis_name="c", subcore_axis_name="s")
    f = pl.kernel(
        _sc_kernel,
        out_type=jax.ShapeDtypeStruct((BATCH, OUT_COLS), jnp.float32),
        mesh=mesh,
        scratch_types=[
            pltpu.VMEM((ROWS_PER_WORKER, IN_COLS), jnp.float32),
            pltpu.VMEM((CHUNK_ROWS, OUT_COLS), jnp.float32),
            pltpu.VMEM((CHUNK_ROWS, OUT_COLS), jnp.float32),
            pltpu.SemaphoreType.DMA,
            pltpu.SemaphoreType.DMA,
        ],
        compiler_params=pltpu.CompilerParams(
            needs_layout_passes=False, use_tc_tiling_on_sc=False
        ),
    )
    return f(x)


# speedup vs baseline: 1.2156x; 1.2156x over previous
import jax
import jax.numpy as jnp
from jax import lax
from jax.experimental import pallas as pl
from jax.experimental.pallas import tpu as pltpu
from jax.experimental.pallas import tpu_sc as plsc

BATCH = 16384
IN_COLS = 100
OUT_COLS = 380
NUM_NONCAT = 60
_CAT_GROUPS = ((4, 60, 20, 60), (8, 80, 10, 140), (16, 90, 10, 220))
_CAT_PARAMS = [
    (src0 + j, out0 + card * j)
    for card, src0, nparams, out0 in _CAT_GROUPS
    for j in range(nparams)
]

NUM_WORKERS = 32
ROWS_PER_WORKER = BATCH // NUM_WORKERS
CHUNK_ROWS = 64
CHUNKS = ROWS_PER_WORKER // CHUNK_ROWS
BLOCKS = CHUNK_ROWS // 16


def _splat(v):
    return jnp.full((16,), v, jnp.int32)


def _block_body(in_v, ob, chunk, blk):
    rows_l = blk * 16 + lax.iota(jnp.int32, 16)
    rows_g = chunk * CHUNK_ROWS + rows_l
    one = jnp.full((16,), 1.0, jnp.float32)
    zero = jnp.zeros((16,), jnp.float32)

    for c in range(NUM_NONCAT):
        v = plsc.load_gather(in_v, [rows_g, _splat(c)])
        plsc.store_scatter(ob, [rows_l, _splat(c)], v)

    @pl.when(chunk >= 2)
    def _clear():
        rows_o = rows_g - 2 * CHUNK_ROWS
        for src, out0 in _CAT_PARAMS:
            vold = plsc.load_gather(in_v, [rows_o, _splat(src)])
            pos = vold.astype(jnp.int32) + out0
            plsc.store_scatter(ob, [rows_l, pos], zero)

    for src, out0 in _CAT_PARAMS:
        v = plsc.load_gather(in_v, [rows_g, _splat(src)])
        pos = v.astype(jnp.int32) + out0
        plsc.store_scatter(ob, [rows_l, pos], one)


def _sc_kernel(x_hbm, out_hbm, in_v, out_v0, out_v1, sem_in, sem0, sem1):
    wid = lax.axis_index("s") * 2 + lax.axis_index("c")
    row0 = wid * ROWS_PER_WORKER

    cp_in = pltpu.async_copy(
        x_hbm.at[pl.ds(row0, ROWS_PER_WORKER)], in_v, sem_in
    )
    zero = jnp.zeros((16,), jnp.float32)
    for ob in (out_v0, out_v1):

        @pl.loop(0, BLOCKS)
        def _z(blk):
            rows_l = blk * 16 + lax.iota(jnp.int32, 16)
            for c in range(NUM_NONCAT, OUT_COLS):
                plsc.store_scatter(ob, [rows_l, _splat(c)], zero)

    cp_in.wait()

    @pl.loop(0, CHUNKS, step=2)
    def _pair(c):
        for phase, (ob, sem) in enumerate(((out_v0, sem0), (out_v1, sem1))):
            chunk = c + phase

            @pl.when(chunk >= 2)
            def _wait_prev():
                pltpu.make_async_copy(
                    ob, out_hbm.at[pl.ds(row0, CHUNK_ROWS)], sem
                ).wait()

            @pl.loop(0, BLOCKS)
            def _blk(blk):
                _block_body(in_v, ob, chunk, blk)

            pltpu.async_copy(
                ob, out_hbm.at[pl.ds(row0 + chunk * CHUNK_ROWS, CHUNK_ROWS)], sem
            )

    pltpu.make_async_copy(out_v0, out_hbm.at[pl.ds(row0, CHUNK_ROWS)], sem0).wait()
    pltpu.make_async_copy(out_v1, out_hbm.at[pl.ds(row0, CHUNK_ROWS)], sem1).wait()


@jax.jit
def kernel(x):
    mesh = plsc.VectorSubcoreMesh(core_axis_name="c", subcore_axis_name="s")
    f = pl.kernel(
        _sc_kernel,
        out_type=jax.ShapeDtypeStruct((BATCH, OUT_COLS), jnp.float32),
        mesh=mesh,
        scratch_types=[
            pltpu.VMEM((ROWS_PER_WORKER, IN_COLS), jnp.float32),
            pltpu.VMEM((CHUNK_ROWS, OUT_COLS), jnp.float32),
            pltpu.VMEM((CHUNK_ROWS, OUT_COLS), jnp.float32),
            pltpu.SemaphoreType.DMA,
            pltpu.SemaphoreType.DMA,
            pltpu.SemaphoreType.DMA,
        ],
        compiler_params=pltpu.CompilerParams(
            needs_layout_passes=False, use_tc_tiling_on_sc=False
        ),
    )
    return f(x)


# speedup vs baseline: 1.5405x; 1.2673x over previous
import jax
import jax.numpy as jnp
from jax import lax
from jax.experimental import pallas as pl
from jax.experimental.pallas import tpu as pltpu
from jax.experimental.pallas import tpu_sc as plsc

BATCH = 16384
IN_COLS = 100
OUT_COLS = 380
NUM_NONCAT = 60
_CAT_GROUPS = ((4, 60, 20, 60), (8, 80, 10, 140), (16, 90, 10, 220))
_CAT_PARAMS = [
    (src0 + j, out0 + card * j)
    for card, src0, nparams, out0 in _CAT_GROUPS
    for j in range(nparams)
]

NUM_WORKERS = 32
ROWS_PER_WORKER = BATCH // NUM_WORKERS
CHUNK_ROWS = 64
CHUNKS = ROWS_PER_WORKER // CHUNK_ROWS
BLOCKS = CHUNK_ROWS // 16


def _splat(v):
    return jnp.full((16,), v, jnp.int32)


_LD_BATCH = 15


def _batched(seq):
    for i in range(0, len(seq), _LD_BATCH):
        yield seq[i : i + _LD_BATCH]


def _block_body(in_v, ob, chunk, blk):
    rows_l = blk * 16 + lax.iota(jnp.int32, 16)
    rows_g = chunk * CHUNK_ROWS + rows_l
    one = jnp.full((16,), 1.0, jnp.float32)
    zero = jnp.zeros((16,), jnp.float32)

    for cols in _batched(list(range(NUM_NONCAT))):
        vals = [plsc.load_gather(in_v, [rows_g, _splat(c)]) for c in cols]
        for c, v in zip(cols, vals):
            plsc.store_scatter(ob, [rows_l, _splat(c)], v)

    @pl.when(chunk >= 2)
    def _clear():
        rows_o = rows_g - 2 * CHUNK_ROWS
        for params in _batched(_CAT_PARAMS):
            olds = [
                plsc.load_gather(in_v, [rows_o, _splat(src)])
                for src, _ in params
            ]
            for (src, out0), vold in zip(params, olds):
                pos = vold.astype(jnp.int32) + out0
                plsc.store_scatter(ob, [rows_l, pos], zero)

    for params in _batched(_CAT_PARAMS):
        vals = [
            plsc.load_gather(in_v, [rows_g, _splat(src)]) for src, _ in params
        ]
        for (src, out0), v in zip(params, vals):
            pos = v.astype(jnp.int32) + out0
            plsc.store_scatter(ob, [rows_l, pos], one)


def _sc_kernel(x_hbm, out_hbm, in_v, out_v0, out_v1, sem_in, sem0, sem1):
    wid = lax.axis_index("s") * 2 + lax.axis_index("c")
    row0 = wid * ROWS_PER_WORKER

    cp_in = pltpu.async_copy(
        x_hbm.at[pl.ds(row0, ROWS_PER_WORKER)], in_v, sem_in
    )
    zero = jnp.zeros((16,), jnp.float32)
    for ob in (out_v0, out_v1):

        @pl.loop(0, BLOCKS)
        def _z(blk):
            rows_l = blk * 16 + lax.iota(jnp.int32, 16)
            for c in range(NUM_NONCAT, OUT_COLS):
                plsc.store_scatter(ob, [rows_l, _splat(c)], zero)

    cp_in.wait()

    @pl.loop(0, CHUNKS, step=2)
    def _pair(c):
        for phase, (ob, sem) in enumerate(((out_v0, sem0), (out_v1, sem1))):
            chunk = c + phase

            @pl.when(chunk >= 2)
            def _wait_prev():
                pltpu.make_async_copy(
                    ob, out_hbm.at[pl.ds(row0, CHUNK_ROWS)], sem
                ).wait()

            @pl.loop(0, BLOCKS)
            def _blk(blk):
                _block_body(in_v, ob, chunk, blk)

            pltpu.async_copy(
                ob, out_hbm.at[pl.ds(row0 + chunk * CHUNK_ROWS, CHUNK_ROWS)], sem
            )

    pltpu.make_async_copy(out_v0, out_hbm.at[pl.ds(row0, CHUNK_ROWS)], sem0).wait()
    pltpu.make_async_copy(out_v1, out_hbm.at[pl.ds(row0, CHUNK_ROWS)], sem1).wait()


@jax.jit
def kernel(x):
    mesh = plsc.VectorSubcoreMesh(core_axis_name="c", subcore_axis_name="s")
    f = pl.kernel(
        _sc_kernel,
        out_type=jax.ShapeDtypeStruct((BATCH, OUT_COLS), jnp.float32),
        mesh=mesh,
        scratch_types=[
            pltpu.VMEM((ROWS_PER_WORKER, IN_COLS), jnp.float32),
            pltpu.VMEM((CHUNK_ROWS, OUT_COLS), jnp.float32),
            pltpu.VMEM((CHUNK_ROWS, OUT_COLS), jnp.float32),
            pltpu.SemaphoreType.DMA,
            pltpu.SemaphoreType.DMA,
            pltpu.SemaphoreType.DMA,
        ],
        compiler_params=pltpu.CompilerParams(
            needs_layout_passes=False, use_tc_tiling_on_sc=False
        ),
    )
    return f(x)


# speedup vs baseline: 1.9288x; 1.2520x over previous
import jax
import jax.numpy as jnp
from jax import lax
from jax.experimental import pallas as pl
from jax.experimental.pallas import tpu as pltpu
from jax.experimental.pallas import tpu_sc as plsc

BATCH = 16384
IN_COLS = 100
OUT_COLS = 380
NUM_NONCAT = 60
_CAT_GROUPS = ((4, 60, 20, 60), (8, 80, 10, 140), (16, 90, 10, 220))
_CAT_PARAMS = [
    (src0 + j, out0 + card * j)
    for card, src0, nparams, out0 in _CAT_GROUPS
    for j in range(nparams)
]

NUM_WORKERS = 32
ROWS_PER_WORKER = BATCH // NUM_WORKERS
CHUNK_ROWS = 64
CHUNKS = ROWS_PER_WORKER // CHUNK_ROWS
BLOCKS = CHUNK_ROWS // 16


def _splat(v):
    return jnp.full((16,), v, jnp.int32)


_LD_BATCH = 15


def _batched(seq):
    for i in range(0, len(seq), _LD_BATCH):
        yield seq[i : i + _LD_BATCH]


def _block_body(in_v, ob, chunk, blk):
    rows_l = blk * 16 + lax.iota(jnp.int32, 16)
    rows_g = chunk * CHUNK_ROWS + rows_l
    one = jnp.full((16,), 1.0, jnp.float32)
    zero = jnp.zeros((16,), jnp.float32)

    for cols in _batched(list(range(NUM_NONCAT))):
        vals = [plsc.load_gather(in_v, [rows_g, _splat(c)]) for c in cols]
        for c, v in zip(cols, vals):
            plsc.store_scatter(ob, [rows_l, _splat(c)], v)

    @pl.when(chunk >= 2)
    def _clear():
        rows_o = rows_g - 2 * CHUNK_ROWS
        for params in _batched(_CAT_PARAMS):
            olds = [
                plsc.load_gather(in_v, [rows_o, _splat(src)])
                for src, _ in params
            ]
            for (src, out0), vold in zip(params, olds):
                pos = vold.astype(jnp.int32) + out0
                plsc.store_scatter(ob, [rows_l, pos], zero)

    for params in _batched(_CAT_PARAMS):
        vals = [
            plsc.load_gather(in_v, [rows_g, _splat(src)]) for src, _ in params
        ]
        for (src, out0), v in zip(params, vals):
            pos = v.astype(jnp.int32) + out0
            plsc.store_scatter(ob, [rows_l, pos], one)


def _sc_kernel(x_hbm, out_hbm, in_v, out_v0, out_v1, sem_in, sem0, sem1):
    wid = lax.axis_index("s") * 2 + lax.axis_index("c")
    row0 = wid * ROWS_PER_WORKER

    cp_in = pltpu.async_copy(
        x_hbm.at[pl.ds(row0, ROWS_PER_WORKER)], in_v, sem_in
    )
    zero = jnp.zeros((16,), jnp.float32)
    for ob in (out_v0, out_v1):

        @pl.loop(0, BLOCKS)
        def _z(blk):
            rows_l = blk * 16 + lax.iota(jnp.int32, 16)
            for c in range(NUM_NONCAT, OUT_COLS):
                plsc.store_scatter(ob, [rows_l, _splat(c)], zero)

    cp_in.wait()

    @pl.loop(0, CHUNKS, step=2)
    def _pair(c):
        for phase, (ob, sem) in enumerate(((out_v0, sem0), (out_v1, sem1))):
            chunk = c + phase

            @pl.when(chunk >= 2)
            def _wait_prev():
                pltpu.make_async_copy(
                    ob, out_hbm.at[pl.ds(row0, CHUNK_ROWS)], sem
                ).wait()

            @pl.loop(0, BLOCKS)
            def _blk(blk):
                _block_body(in_v, ob, chunk, blk)

            pltpu.async_copy(
                ob, out_hbm.at[pl.ds(row0 + chunk * CHUNK_ROWS, CHUNK_ROWS)], sem
            )

    pltpu.make_async_copy(out_v0, out_hbm.at[pl.ds(row0, CHUNK_ROWS)], sem0).wait()
    pltpu.make_async_copy(out_v1, out_hbm.at[pl.ds(row0, CHUNK_ROWS)], sem1).wait()


@jax.jit
def kernel(x):
    mesh = plsc.VectorSubcoreMesh(core_axis_name="c", subcore_axis_name="s")
    f = pl.kernel(
        _sc_kernel,
        out_type=jax.ShapeDtypeStruct((BATCH, OUT_COLS), jnp.float32),
        mesh=mesh,
        scratch_types=[
            pltpu.VMEM((ROWS_PER_WORKER, IN_COLS), jnp.float32),
            pltpu.VMEM((CHUNK_ROWS, OUT_COLS), jnp.float32),
            pltpu.VMEM((CHUNK_ROWS, OUT_COLS), jnp.float32),
            pltpu.SemaphoreType.DMA,
            pltpu.SemaphoreType.DMA,
            pltpu.SemaphoreType.DMA,
        ],
        compiler_params=pltpu.CompilerParams(
            needs_layout_passes=False, use_tc_tiling_on_sc=True
        ),
    )
    return f(x)


# speedup vs baseline: 4.4620x; 2.3134x over previous
import jax
import jax.numpy as jnp
from jax import lax
from jax.experimental import pallas as pl
from jax.experimental.pallas import tpu as pltpu
from jax.experimental.pallas import tpu_sc as plsc

BATCH = 16384
IN_COLS = 100
OUT_COLS = 380
NUM_NONCAT = 60

NUM_WORKERS = 32
ROWS_PER_WORKER = BATCH // NUM_WORKERS
CHUNK_ROWS = 64
CHUNKS = ROWS_PER_WORKER // CHUNK_ROWS

_O_CARD4 = 61
_O_K8 = 77
_O_CARD8 = 80
_O_TAIL = 84


def _row_body(in_v, ob, rg, r, consts):
    (lane, b4, b8, b16, cls4, cls8, cls16, pat3, mask3, one, zero) = consts

    ld = {}
    for o in (0, 16, 32, 48, _O_CARD4, _O_K8, _O_CARD8, _O_TAIL):
        ld[o] = in_v[rg, pl.ds(o, 16)]

    def perm(v, idx):
        dn = lax.GatherDimensionNumbers(
            offset_dims=(), collapsed_slice_dims=(0,), start_index_map=(0,)
        )
        return lax.gather(
            v,
            idx[:, None],
            dimension_numbers=dn,
            slice_sizes=(1,),
            mode=lax.GatherScatterMode.PROMISE_IN_BOUNDS,
        )

    def onehot(src_o, pat, cls):
        s = perm(ld[src_o], pat)
        return jnp.where(s == cls, one, zero)

    ob[r, pl.ds(0, 16)] = ld[0]
    ob[r, pl.ds(16, 16)] = ld[16]
    ob[r, pl.ds(32, 16)] = ld[32]
    s3 = perm(ld[48], pat3)
    oh3 = jnp.where(s3 == cls4, one, zero)
    ob[r, pl.ds(48, 16)] = jnp.where(mask3, s3, oh3)
    for k in range(4, 8):
        pat = (60 + (16 * k - 60) // 4 - _O_CARD4) + b4
        ob[r, pl.ds(16 * k, 16)] = onehot(_O_CARD4, pat, cls4)
    ob[r, pl.ds(128, 16)] = onehot(_O_K8, b4, cls4)
    for k in range(9, 13):
        pat = (80 + (16 * k - 144) // 8 - _O_CARD8) + b8
        ob[r, pl.ds(16 * k, 16)] = onehot(_O_CARD8, pat, cls8)
    ob[r, pl.ds(208, 16)] = onehot(_O_TAIL, 4 + b8, cls8)
    for k in range(14, 23):
        pat = (90 + (16 * k - 224) // 16 - _O_TAIL) + b16
        ob[r, pl.ds(16 * k, 16)] = onehot(_O_TAIL, pat, cls16)
    pat23 = jnp.minimum((90 + (16 * 23 - 224) // 16 - _O_TAIL) + b16, 15)
    v23 = onehot(_O_TAIL, pat23, cls16)
    plsc.store_scatter(
        ob, [jnp.full((16,), r, jnp.int32), 368 + lane], v23, mask=lane < 12
    )


def _sc_kernel(x_hbm, out_hbm, in_v, out_v0, out_v1, sem_in, sem0, sem1):
    wid = lax.axis_index("s") * 2 + lax.axis_index("c")
    row0 = wid * ROWS_PER_WORKER

    pltpu.async_copy(x_hbm.at[pl.ds(row0, ROWS_PER_WORKER)], in_v, sem_in).wait()

    lane = lax.iota(jnp.int32, 16)
    consts = (
        lane,
        lane >> 2,
        (lane + 4) >> 3,
        (lane + 4) >> 4,
        (lane & 3).astype(jnp.float32),
        ((lane + 4) & 7).astype(jnp.float32),
        ((lane + 4) & 15).astype(jnp.float32),
        jnp.minimum(lane, 12),
        lane < 12,
        jnp.full((16,), 1.0, jnp.float32),
        jnp.zeros((16,), jnp.float32),
    )

    @pl.loop(0, CHUNKS, step=2)
    def _pair(c):
        for phase, (ob, sem) in enumerate(((out_v0, sem0), (out_v1, sem1))):
            chunk = c + phase

            @pl.when(chunk >= 2)
            def _wait_prev():
                pltpu.make_async_copy(
                    ob, out_hbm.at[pl.ds(row0, CHUNK_ROWS)], sem
                ).wait()

            @pl.loop(0, CHUNK_ROWS)
            def _row(r):
                _row_body(in_v, ob, chunk * CHUNK_ROWS + r, r, consts)

            pltpu.async_copy(
                ob, out_hbm.at[pl.ds(row0 + chunk * CHUNK_ROWS, CHUNK_ROWS)], sem
            )

    pltpu.make_async_copy(out_v0, out_hbm.at[pl.ds(row0, CHUNK_ROWS)], sem0).wait()
    pltpu.make_async_copy(out_v1, out_hbm.at[pl.ds(row0, CHUNK_ROWS)], sem1).wait()


@jax.jit
def kernel(x):
    mesh = plsc.VectorSubcoreMesh(core_axis_name="c", subcore_axis_name="s")
    f = pl.kernel(
        _sc_kernel,
        out_type=jax.ShapeDtypeStruct((BATCH, OUT_COLS), jnp.float32),
        mesh=mesh,
        scratch_types=[
            pltpu.VMEM((ROWS_PER_WORKER, IN_COLS), jnp.float32),
            pltpu.VMEM((CHUNK_ROWS, OUT_COLS), jnp.float32),
            pltpu.VMEM((CHUNK_ROWS, OUT_COLS), jnp.float32),
            pltpu.SemaphoreType.DMA,
            pltpu.SemaphoreType.DMA,
            pltpu.SemaphoreType.DMA,
        ],
        compiler_params=pltpu.CompilerParams(
            needs_layout_passes=False, use_tc_tiling_on_sc=True
        ),
    )
    return f(x)
